# initial kernel scaffold (unmeasured)
import jax
import jax.numpy as jnp
from jax import lax
from jax.experimental import pallas as pl
from jax.experimental.pallas import tpu as pltpu

N_DEV = 32


def kernel(x, w_mat, scale_x, scale_w):
    m_per, k = x.shape
    n = w_mat.shape[1]
    n_per = n // N_DEV

    def body(x_ref, w_ref, sx_ref, sw_ref, out_ref, acc_ref, send_sems, recv_sems):
        me = lax.axis_index("i")

        acc = jnp.dot(x_ref[:, :], w_ref[:, :], preferred_element_type=jnp.float32)
        acc_ref[:, :] = acc * (sx_ref[0] * sw_ref[0])

        out_ref[pl.ds(me * m_per, m_per), :] = acc_ref[:, pl.ds(me * n_per, n_per)]

        rdmas = []
        for h in range(1, N_DEV):
            dst = lax.rem(me + h, N_DEV)
            rdma = pltpu.make_async_remote_copy(
                src_ref=acc_ref.at[:, pl.ds(dst * n_per, n_per)],
                dst_ref=out_ref.at[pl.ds(me * m_per, m_per), :],
                send_sem=send_sems.at[h - 1],
                recv_sem=recv_sems.at[h - 1],
                device_id=(dst,),
                device_id_type=pl.DeviceIdType.MESH,
            )
            rdma.start()
            rdmas.append(rdma)
        for rdma in rdmas:
            rdma.wait()

    return pl.pallas_call(
        body,
        out_shape=jax.ShapeDtypeStruct((N_DEV * m_per, n_per), jnp.float32),
        in_specs=[
            pl.BlockSpec(memory_space=pltpu.VMEM),
            pl.BlockSpec(memory_space=pltpu.VMEM),
            pl.BlockSpec(memory_space=pltpu.SMEM),
            pl.BlockSpec(memory_space=pltpu.SMEM),
        ],
        out_specs=pl.BlockSpec(memory_space=pltpu.VMEM),
        scratch_shapes=[
            pltpu.VMEM((m_per, n), jnp.float32),
            pltpu.SemaphoreType.DMA((N_DEV - 1,)),
            pltpu.SemaphoreType.DMA((N_DEV - 1,)),
        ],
        compiler_params=pltpu.CompilerParams(collective_id=0),
    )(x, w_mat, scale_x, scale_w)


# baseline (device time: 54886 ns/iter reference)
import jax
import jax.numpy as jnp
from jax import lax
from jax.experimental import pallas as pl
from jax.experimental.pallas import tpu as pltpu

N_DEV = 32


def kernel(x, w_mat, scale_x, scale_w):
    m_per, k = x.shape
    n = w_mat.shape[1]
    n_per = n // N_DEV

    def body(x_ref, w_ref, sx_ref, sw_ref, out_ref, comm_ref, send_sems, recv_sems):
        me = lax.axis_index("i")

        acc = jnp.dot(x_ref[:, :], w_ref[:, :], preferred_element_type=jnp.float32)
        acc = acc * (sx_ref[0] * sw_ref[0])

        for j in range(N_DEV):
            comm_ref[j, :, :] = acc[:, j * n_per:(j + 1) * n_per]

        my_rows = pl.multiple_of(me * m_per, m_per)
        out_ref[pl.ds(my_rows, m_per), :] = comm_ref[me]

        rdmas = []
        for h in range(1, N_DEV):
            dst = lax.rem(me + h, N_DEV)
            rdma = pltpu.make_async_remote_copy(
                src_ref=comm_ref.at[dst],
                dst_ref=out_ref.at[pl.ds(my_rows, m_per)],
                send_sem=send_sems.at[h - 1],
                recv_sem=recv_sems.at[h - 1],
                device_id=(dst,),
                device_id_type=pl.DeviceIdType.MESH,
            )
            rdma.start()
            rdmas.append(rdma)
        for rdma in rdmas:
            rdma.wait()

    return pl.pallas_call(
        body,
        out_shape=jax.ShapeDtypeStruct((N_DEV * m_per, n_per), jnp.float32),
        in_specs=[
            pl.BlockSpec(memory_space=pltpu.VMEM),
            pl.BlockSpec(memory_space=pltpu.VMEM),
            pl.BlockSpec(memory_space=pltpu.SMEM),
            pl.BlockSpec(memory_space=pltpu.SMEM),
        ],
        out_specs=pl.BlockSpec(memory_space=pltpu.VMEM),
        scratch_shapes=[
            pltpu.VMEM((N_DEV, m_per, n_per), jnp.float32),
            pltpu.SemaphoreType.DMA((N_DEV - 1,)),
            pltpu.SemaphoreType.DMA((N_DEV - 1,)),
        ],
        compiler_params=pltpu.CompilerParams(vmem_limit_bytes=100 * 1024 * 1024),
    )(x, w_mat, scale_x, scale_w)
